# baseline (speedup 1.0000x reference)
import jax
import jax.numpy as jnp
import numpy as np
from jax.experimental import pallas as pl
from jax.experimental.pallas import tpu as pltpu

_K = 64
_VT = 2048


def _dist_body(x_ref, b_ref, out_ref):
    x = x_ref[...]
    bt = b_ref[...]
    sq_x = jnp.sum(x * x, axis=1, keepdims=True)
    sq_b = jnp.sum(bt * bt, axis=1)
    prod = jax.lax.dot_general(x, bt, (((1,), (1,)), ((), ())),
                               preferred_element_type=jnp.float32)
    out_ref[...] = sq_x + sq_b[None, :] - 2.0 * prod


def kernel(input, target, b):
    n, d = input.shape
    v = b.shape[0]
    n_tiles = (v + _VT - 1) // _VT
    v_pad = n_tiles * _VT
    b_p = jnp.pad(b, ((0, v_pad - v), (0, 0)))
    dists = pl.pallas_call(
        _dist_body,
        grid=(n_tiles,),
        in_specs=[
            pl.BlockSpec((n, d), lambda i: (0, 0)),
            pl.BlockSpec((_VT, d), lambda i: (i, 0)),
        ],
        out_specs=pl.BlockSpec((n, _VT), lambda i: (0, i)),
        out_shape=jax.ShapeDtypeStruct((n, v_pad), jnp.float32),
    )(input, b_p)
    neg_topk, _ = jax.lax.top_k(-dists[:, :v], _K)
    return -neg_topk


# speedup vs baseline: 22.8883x; 22.8883x over previous
import functools

import jax
import jax.numpy as jnp
from jax.experimental import pallas as pl
from jax.experimental.pallas import tpu as pltpu

_K = 64
_VT = 2048
_BIG = 3.0e38


def _stage(a, j, s, invert=False):
    n, Q = a.shape
    g = n // (2 * j)
    ar = a.reshape(g, 2, j, Q)
    A = ar[:, 0]
    B = ar[:, 1]
    mn = jnp.minimum(A, B)
    mx = jnp.maximum(A, B)
    q = jax.lax.broadcasted_iota(jnp.int32, (g, 1, 1), 0)
    asc = ((q * (2 * j)) % (2 * s)) < s
    if invert:
        asc = jnp.logical_not(asc)
    newA = jnp.where(asc, mn, mx)
    newB = jnp.where(asc, mx, mn)
    return jnp.stack([newA, newB], axis=1).reshape(n, Q)


def _sort_runs(a, run):
    s = 2
    while s <= run:
        j = s >> 1
        while j >= 1:
            a = _stage(a, j, s)
            j >>= 1
        s <<= 1
    return a


def _cleanup(a, run, invert=False):
    j = run >> 1
    while j >= 1:
        a = _stage(a, j, run, invert)
        j >>= 1
    return a


def _pair_min(a):
    n, Q = a.shape
    ar = a.reshape(n // (2 * _K), 2, _K, Q)
    return jnp.minimum(ar[:, 0], ar[:, 1]).reshape(n // 2, Q)


def _topk_body(v_real, x_ref, b_ref, out_ref, s_ref):
    i = pl.program_id(0)
    nt = pl.num_programs(0)
    Q = x_ref.shape[0]

    @pl.when(i == 0)
    def _init():
        s_ref[...] = jnp.full((_K, Q), _BIG, jnp.float32)

    x = x_ref[...]
    bt = b_ref[...]
    sq_x = jnp.sum(x * x, axis=1)
    sq_b = jnp.sum(bt * bt, axis=1)
    prod = jax.lax.dot_general(bt, x, (((1,), (1,)), ((), ())),
                               preferred_element_type=jnp.float32)
    d = sq_b[:, None] + sq_x[None, :] - 2.0 * prod

    row = jax.lax.broadcasted_iota(jnp.int32, (_VT, 1), 0) + i * _VT
    d = jnp.where(row >= v_real, _BIG, d)

    d = _sort_runs(d, _K)
    n = _VT
    while n > _K:
        d = _pair_min(d)
        n //= 2
        d = _cleanup(d, _K, invert=(n == _K))

    merged = jnp.minimum(s_ref[...], d)
    s_ref[...] = _cleanup(merged, _K)

    @pl.when(i == nt - 1)
    def _done():
        out_ref[...] = s_ref[...]


def kernel(input, target, b):
    n, dim = input.shape
    v = b.shape[0]
    nt = (v + _VT - 1) // _VT
    v_pad = nt * _VT
    b_p = jnp.pad(b, ((0, v_pad - v), (0, 0)))
    out = pl.pallas_call(
        functools.partial(_topk_body, v),
        grid=(nt,),
        in_specs=[
            pl.BlockSpec((n, dim), lambda i: (0, 0)),
            pl.BlockSpec((_VT, dim), lambda i: (i, 0)),
        ],
        out_specs=pl.BlockSpec((_K, n), lambda i: (0, 0)),
        out_shape=jax.ShapeDtypeStruct((_K, n), jnp.float32),
        scratch_shapes=[pltpu.VMEM((_K, n), jnp.float32)],
        compiler_params=pltpu.CompilerParams(
            dimension_semantics=("arbitrary",),
        ),
    )(input, b_p)
    return out.T
